# baseline (speedup 1.0000x reference)
import functools

import jax
import jax.numpy as jnp
from jax import lax
from jax.experimental import pallas as pl
from jax.experimental.pallas import tpu as pltpu
from jax.experimental.pallas import tpu_sc as plsc

N_NODES = 10000
HIDDEN = 64
Z_DIM = 32
T_SEQ = 20

NC = 2
NS = 16
NW = NC * NS
CHUNK = 128
NP = 10240
ROWS_PER_TILE = NP // NS

BLK = 1000
ABLK = 400


def _make_segsum(width, nc0, nc1, with_deg):
  mesh = plsc.VectorSubcoreMesh(core_axis_name="c", subcore_axis_name="s")
  ncmax = max(nc0, nc1)

  out_type = [jax.ShapeDtypeStruct((NC, NP, width), jnp.float32)]
  scratch = [
      pltpu.VMEM((ncmax, CHUNK), jnp.int32),
      pltpu.VMEM((ncmax, CHUNK), jnp.int32),
      pltpu.VMEM((2, CHUNK, width), jnp.float32),
      pltpu.VMEM_SHARED((NP, width), jnp.float32),
      pltpu.SemaphoreType.DMA, pltpu.SemaphoreType.DMA,
      pltpu.SemaphoreType.DMA, pltpu.SemaphoreType.DMA,
  ]
  if with_deg:
    out_type.append(jax.ShapeDtypeStruct((NC, NP), jnp.float32))
    scratch += [
        pltpu.VMEM((CHUNK,), jnp.float32),
        pltpu.VMEM((ROWS_PER_TILE,), jnp.float32),
        pltpu.VMEM_SHARED((NP,), jnp.float32),
        pltpu.SemaphoreType.DMA,
    ]

  def body(table, src2, dst2, *rest):
    if with_deg:
      (out_acc, out_deg, src_v, dst_v, rows_v, acc_s, sg0, sg1, ss0, ss1,
       ones_v, zdeg_v, deg_s, semd) = rest
    else:
      out_acc, src_v, dst_v, rows_v, acc_s, sg0, sg1, ss0, ss1 = rest
    semg = (sg0, sg1)
    sems = (ss0, ss1)
    c = lax.axis_index("c")
    s = lax.axis_index("s")

    def zrow(i, carry):
      for j in range(width // 16):
        rows_v[0, i, pl.ds(j * 16, 16)] = jnp.zeros((16,), jnp.float32)
      return carry
    lax.fori_loop(0, CHUNK, zrow, 0)
    for i in range(ROWS_PER_TILE // CHUNK):
      pltpu.sync_copy(rows_v.at[0],
                      acc_s.at[pl.ds(s * ROWS_PER_TILE + i * CHUNK, CHUNK)])
    if with_deg:
      def zdeg(i, carry):
        zdeg_v[pl.ds(i * 16, 16)] = jnp.zeros((16,), jnp.float32)
        ones_v[pl.ds((i % (CHUNK // 16)) * 16, 16)] = jnp.ones((16,), jnp.float32)
        return carry
      lax.fori_loop(0, ROWS_PER_TILE // 16, zdeg, 0)
      pltpu.sync_copy(zdeg_v, deg_s.at[pl.ds(s * ROWS_PER_TILE, ROWS_PER_TILE)])
    plsc.subcore_barrier()

    def scatter_phase(base, count):
      pltpu.sync_copy(src2.at[pl.ds(base, count)], src_v.at[pl.ds(0, count)])
      pltpu.sync_copy(dst2.at[pl.ds(base, count)], dst_v.at[pl.ds(0, count)])

      for b in range(2):
        pltpu.async_copy(table.at[src_v.at[b]], rows_v.at[b], semg[b])

      @pl.loop(0, count, step=2)
      def chunk_body(i):
        for b in range(2):
          cix = i + b
          pltpu.make_async_copy(table.at[src_v.at[cix]], rows_v.at[b],
                                semg[b]).wait()
          pltpu.async_copy(rows_v.at[b], acc_s.at[dst_v.at[cix]], sems[b],
                           add=True)
          if with_deg:
            pltpu.async_copy(ones_v, deg_s.at[dst_v.at[cix]], semd, add=True)
        for b in range(2):
          cix = i + 2 + b

          @pl.when(cix < count)
          def _():
            pltpu.make_async_copy(rows_v.at[b], acc_s.at[dst_v.at[cix]],
                                  sems[b]).wait()
            pltpu.async_copy(table.at[src_v.at[cix]], rows_v.at[b], semg[b])

      for b in range(2):
        pltpu.make_async_copy(rows_v.at[b], acc_s.at[dst_v.at[b]],
                              sems[b]).wait()
      if with_deg:
        def ddrain(i, carry):
          pltpu.make_async_copy(ones_v, deg_s.at[dst_v.at[i]], semd).wait()
          return carry
        lax.fori_loop(0, count, ddrain, 0)

    @pl.when(c == 0)
    def _():
      scatter_phase(s * nc0, nc0)

    @pl.when(c != 0)
    def _():
      scatter_phase(NS * nc0 + s * nc1, nc1)

    plsc.subcore_barrier()

    row0 = s * ROWS_PER_TILE
    pltpu.sync_copy(acc_s.at[pl.ds(row0, ROWS_PER_TILE)],
                    out_acc.at[c, pl.ds(row0, ROWS_PER_TILE)])
    if with_deg:
      pltpu.sync_copy(deg_s.at[pl.ds(row0, ROWS_PER_TILE)],
                      out_deg.at[c, pl.ds(row0, ROWS_PER_TILE)])

  return pl.kernel(body, out_type=out_type, mesh=mesh, scratch_types=scratch,
                   compiler_params=pltpu.CompilerParams(use_tc_tiling_on_sc=False))


def _gru_cell(xs2, h, wt, wh, b):
  g = (jnp.dot(xs2, wt, preferred_element_type=jnp.float32)
       + jnp.dot(h, wh, preferred_element_type=jnp.float32) + b)
  rz = jax.nn.sigmoid(g[:, :2 * HIDDEN])
  r = rz[:, :HIDDEN]
  zg = rz[:, HIDDEN:]
  n = jnp.tanh(g[:, 2 * HIDDEN:3 * HIDDEN] + r * g[:, 3 * HIDDEN:])
  return n + zg * (h - n)


def _gru_wcomb(w_ih, w_hh, b_ih, b_hh):
  wih, whh = w_ih.T, w_hh.T
  d = wih.shape[0]
  h2 = 2 * HIDDEN
  wih4 = jnp.concatenate(
      [wih[:, :h2], wih[:, h2:], jnp.zeros((d, HIDDEN), jnp.float32)], axis=1)
  eye = jnp.eye(T_SEQ, dtype=jnp.float32)
  wt = (eye[:, :, None, None] * wih4[None, None, :, :]).reshape(
      T_SEQ, T_SEQ * d, 4 * HIDDEN)
  wh = jnp.concatenate(
      [whh[:, :h2], jnp.zeros((HIDDEN, HIDDEN), jnp.float32), whh[:, h2:]],
      axis=1)
  b = jnp.concatenate([b_ih[:h2] + b_hh[:h2], b_ih[h2:], b_hh[h2:]])
  return wt, wh, b.reshape(1, -1)


def _pre_body(x_ref, w1_ref, table_ref, xr_ref):
  x = x_ref[...]
  y = jnp.dot(x, w1_ref[...], preferred_element_type=jnp.float32)
  table_ref[...] = y[:, :HIDDEN]
  xr_ref[...] = y[:, HIDDEN:]


def _gru_body(xs_ref, wtf_ref, whf_ref, bf_ref, wtb_ref, whb_ref, bb_ref,
              wfc_ref, bfc_ref, zmul_ref):
  xs2 = xs_ref[...]
  b = xs2.shape[0]
  hf = jnp.zeros((b, HIDDEN), jnp.float32)
  hb = jnp.zeros((b, HIDDEN), jnp.float32)
  whf, bf = whf_ref[...], bf_ref[...]
  whb, bb = whb_ref[...], bb_ref[...]
  for t in range(T_SEQ):
    hf = _gru_cell(xs2, hf, wtf_ref[t], whf, bf)
    hb = _gru_cell(xs2, hb, wtb_ref[T_SEQ - 1 - t], whb, bb)
  hcat = jnp.concatenate([hf, hb], axis=1)
  zmul_ref[...] = (jnp.dot(hcat, wfc_ref[...], preferred_element_type=jnp.float32)
                   + bfc_ref[...])


def _combine1_body(a0_ref, a1_ref, d0_ref, d1_ref, xr_ref, bl1_ref, w2_ref,
                   table2_ref, hr_ref):
  deg = d0_ref[...] + d1_ref[...]
  inv = 1.0 / jnp.maximum(deg, 1.0)
  h = jnp.maximum((a0_ref[...] + a1_ref[...]) * inv + bl1_ref[...] + xr_ref[...],
                  0.0)
  y = jnp.dot(h, w2_ref[...], preferred_element_type=jnp.float32)
  table2_ref[...] = y[:, :Z_DIM]
  hr_ref[...] = y[:, Z_DIM:]


def _fuse_body(a0_ref, a1_ref, d0_ref, d1_ref, hr_ref, bl2_ref, zmul_ref,
               wfus_ref, bfus_ref, cen_ref, z_ref, q_ref):
  deg = d0_ref[...] + d1_ref[...]
  inv = 1.0 / jnp.maximum(deg, 1.0)
  zg = (a0_ref[...] + a1_ref[...]) * inv + bl2_ref[...] + hr_ref[...]
  comb = jnp.concatenate([zg, zmul_ref[...]], axis=1)
  z = jnp.dot(comb, wfus_ref[...], preferred_element_type=jnp.float32) + bfus_ref[...]
  z_ref[...] = z
  cen = cen_ref[...]
  zc = lax.dot_general(z, cen, (((1,), (1,)), ((), ())),
                       preferred_element_type=jnp.float32)
  z2 = jnp.sum(z * z, axis=1, keepdims=True)
  c2 = jnp.sum(cen * cen, axis=1)[None, :]
  d2 = z2 + c2 - 2.0 * zc
  q = 1.0 / (1.0 + d2)
  q_ref[...] = q / jnp.sum(q, axis=1, keepdims=True)


def _adj_body(zi_ref, zj_ref, out_ref):
  out_ref[...] = jax.nn.sigmoid(
      lax.dot_general(zi_ref[...], zj_ref[...], (((1,), (1,)), ((), ())),
                      preferred_element_type=jnp.float32))


def kernel(x_static, edge_index, x_seq, Wl1, bl1, Wr1, Wl2, bl2, Wr2,
           W_ih_f, W_hh_f, b_ih_f, b_hh_f, W_ih_b, W_hh_b, b_ih_b, b_hh_b,
           W_fc, b_fc, W_fus, b_fus, centers):
  n = x_static.shape[0]
  e = edge_index.shape[1]
  ncl = centers.shape[0]
  grid = n // BLK

  w1cat = jnp.concatenate([Wl1, Wr1], axis=0).T
  w2cat = jnp.concatenate([Wl2, Wr2], axis=0).T
  src = edge_index[0].astype(jnp.int32)
  dst = edge_index[1].astype(jnp.int32)
  ct = -(-e // CHUNK)
  nc0 = max(2, 2 * round(ct * 0.52 / NS / 2))
  nc1 = max(2, 2 * (-(-max(ct - NS * nc0, 0) // (2 * NS))))
  ep = NS * (nc0 + nc1) * CHUNK
  src2 = jnp.concatenate([src, jnp.zeros((ep - e,), jnp.int32)]).reshape(
      -1, CHUNK)
  dst2 = jnp.concatenate([dst, jnp.full((ep - e,), n, jnp.int32)]).reshape(
      -1, CHUNK)

  full = lambda *shape: pl.BlockSpec(shape, lambda i: (0,) * len(shape))
  rowblk = lambda w: pl.BlockSpec((BLK, w), lambda i: (i, 0))
  partblk = lambda w: pl.BlockSpec((None, BLK, w), lambda i, _c=0: (_c, i, 0))

  table1, xr = pl.pallas_call(
      _pre_body,
      grid=(grid,),
      in_specs=[rowblk(128), full(128, 128)],
      out_specs=[rowblk(HIDDEN), rowblk(HIDDEN)],
      out_shape=[
          jax.ShapeDtypeStruct((n, HIDDEN), jnp.float32),
          jax.ShapeDtypeStruct((n, HIDDEN), jnp.float32),
      ],
  )(x_static, w1cat)

  agg1, degp = _make_segsum(HIDDEN, nc0, nc1, True)(table1, src2, dst2)
  degp = degp.reshape(NC, NP, 1)

  seq_d = x_seq.shape[2]
  wtf, whf, bgf = _gru_wcomb(W_ih_f, W_hh_f, b_ih_f, b_hh_f)
  wtb, whb, bgb = _gru_wcomb(W_ih_b, W_hh_b, b_ih_b, b_hh_b)
  td = T_SEQ * seq_d
  zmul = pl.pallas_call(
      _gru_body,
      grid=(grid,),
      in_specs=[
          rowblk(td),
          full(T_SEQ, td, 4 * HIDDEN), full(HIDDEN, 4 * HIDDEN),
          full(1, 4 * HIDDEN),
          full(T_SEQ, td, 4 * HIDDEN), full(HIDDEN, 4 * HIDDEN),
          full(1, 4 * HIDDEN),
          full(2 * HIDDEN, Z_DIM), full(1, Z_DIM),
      ],
      out_specs=rowblk(Z_DIM),
      out_shape=jax.ShapeDtypeStruct((n, Z_DIM), jnp.float32),
  )(x_seq.reshape(n, td), wtf, whf, bgf, wtb, whb, bgb,
    W_fc.T, b_fc.reshape(1, -1))

  part64 = [pl.BlockSpec((None, BLK, HIDDEN), lambda i: (0, i, 0)),
            pl.BlockSpec((None, BLK, HIDDEN), lambda i: (1, i, 0))]
  partd = [pl.BlockSpec((None, BLK, 1), lambda i: (0, i, 0)),
           pl.BlockSpec((None, BLK, 1), lambda i: (1, i, 0))]
  table2, hr = pl.pallas_call(
      _combine1_body,
      grid=(grid,),
      in_specs=part64 + partd + [rowblk(HIDDEN), full(1, HIDDEN),
                                 full(HIDDEN, HIDDEN)],
      out_specs=[rowblk(Z_DIM), rowblk(Z_DIM)],
      out_shape=[
          jax.ShapeDtypeStruct((n, Z_DIM), jnp.float32),
          jax.ShapeDtypeStruct((n, Z_DIM), jnp.float32),
      ],
  )(agg1, agg1, degp, degp, xr, bl1.reshape(1, -1), w2cat)

  (agg2,) = _make_segsum(Z_DIM, nc0, nc1, False)(table2, src2, dst2)

  part32 = [pl.BlockSpec((None, BLK, Z_DIM), lambda i: (0, i, 0)),
            pl.BlockSpec((None, BLK, Z_DIM), lambda i: (1, i, 0))]
  z, q = pl.pallas_call(
      _fuse_body,
      grid=(grid,),
      in_specs=part32 + partd + [rowblk(Z_DIM), full(1, Z_DIM), rowblk(Z_DIM),
                                 full(2 * Z_DIM, Z_DIM), full(1, Z_DIM),
                                 full(ncl, Z_DIM)],
      out_specs=[rowblk(Z_DIM), rowblk(ncl)],
      out_shape=[
          jax.ShapeDtypeStruct((n, Z_DIM), jnp.float32),
          jax.ShapeDtypeStruct((n, ncl), jnp.float32),
      ],
  )(agg2, agg2, degp, degp, hr, bl2.reshape(1, -1), zmul,
    W_fus.T, b_fus.reshape(1, -1), centers)

  adj = pl.pallas_call(
      _adj_body,
      grid=(n // ABLK,),
      in_specs=[pl.BlockSpec((ABLK, Z_DIM), lambda i: (i, 0)),
                pl.BlockSpec((n, Z_DIM), lambda i: (0, 0))],
      out_specs=pl.BlockSpec((ABLK, n), lambda i: (i, 0)),
      out_shape=jax.ShapeDtypeStruct((n, n), jnp.float32),
  )(z, z)

  return (z, adj, q)


# speedup vs baseline: 1.0096x; 1.0096x over previous
import functools

import jax
import jax.numpy as jnp
from jax import lax
from jax.experimental import pallas as pl
from jax.experimental.pallas import tpu as pltpu
from jax.experimental.pallas import tpu_sc as plsc

N_NODES = 10000
HIDDEN = 64
Z_DIM = 32
T_SEQ = 20

NC = 2
NS = 16
NW = NC * NS
CHUNK = 128
NP = 10240
ROWS_PER_TILE = NP // NS

BLK = 1000
ABLK = 400


def _make_segsum(width, nc0, nc1, with_deg):
  mesh = plsc.VectorSubcoreMesh(core_axis_name="c", subcore_axis_name="s")
  ncmax = max(nc0, nc1)

  out_type = [jax.ShapeDtypeStruct((NC, NP, width), jnp.float32)]
  scratch = [
      pltpu.VMEM((ncmax, CHUNK), jnp.int32),
      pltpu.VMEM((ncmax, CHUNK), jnp.int32),
      pltpu.VMEM((2, CHUNK, width), jnp.float32),
      pltpu.VMEM_SHARED((NP, width), jnp.float32),
      pltpu.SemaphoreType.DMA, pltpu.SemaphoreType.DMA,
      pltpu.SemaphoreType.DMA, pltpu.SemaphoreType.DMA,
  ]
  if with_deg:
    out_type.append(jax.ShapeDtypeStruct((NC, NP), jnp.float32))
    scratch += [
        pltpu.VMEM((CHUNK,), jnp.float32),
        pltpu.VMEM((ROWS_PER_TILE,), jnp.float32),
        pltpu.VMEM_SHARED((NP,), jnp.float32),
        pltpu.SemaphoreType.DMA,
    ]

  def body(table, src2, dst2, *rest):
    if with_deg:
      (out_acc, out_deg, src_v, dst_v, rows_v, acc_s, sg0, sg1, ss0, ss1,
       ones_v, zdeg_v, deg_s, semd) = rest
    else:
      out_acc, src_v, dst_v, rows_v, acc_s, sg0, sg1, ss0, ss1 = rest
    semg = (sg0, sg1)
    sems = (ss0, ss1)
    c = lax.axis_index("c")
    s = lax.axis_index("s")

    def zrow(i, carry):
      for j in range(width // 16):
        rows_v[0, i, pl.ds(j * 16, 16)] = jnp.zeros((16,), jnp.float32)
      return carry
    lax.fori_loop(0, CHUNK, zrow, 0)
    for i in range(ROWS_PER_TILE // CHUNK):
      pltpu.sync_copy(rows_v.at[0],
                      acc_s.at[pl.ds(s * ROWS_PER_TILE + i * CHUNK, CHUNK)])
    if with_deg:
      def zdeg(i, carry):
        zdeg_v[pl.ds(i * 16, 16)] = jnp.zeros((16,), jnp.float32)
        ones_v[pl.ds((i % (CHUNK // 16)) * 16, 16)] = jnp.ones((16,), jnp.float32)
        return carry
      lax.fori_loop(0, ROWS_PER_TILE // 16, zdeg, 0)
      pltpu.sync_copy(zdeg_v, deg_s.at[pl.ds(s * ROWS_PER_TILE, ROWS_PER_TILE)])
    plsc.subcore_barrier()

    def scatter_phase(base, count):
      pltpu.sync_copy(src2.at[pl.ds(base, count)], src_v.at[pl.ds(0, count)])
      pltpu.sync_copy(dst2.at[pl.ds(base, count)], dst_v.at[pl.ds(0, count)])

      for b in range(2):
        pltpu.async_copy(table.at[src_v.at[b]], rows_v.at[b], semg[b])

      @pl.loop(0, count, step=2)
      def chunk_body(i):
        for b in range(2):
          cix = i + b
          pltpu.make_async_copy(table.at[src_v.at[cix]], rows_v.at[b],
                                semg[b]).wait()
          pltpu.async_copy(rows_v.at[b], acc_s.at[dst_v.at[cix]], sems[b],
                           add=True)
          if with_deg:
            pltpu.async_copy(ones_v, deg_s.at[dst_v.at[cix]], semd, add=True)
        for b in range(2):
          cix = i + 2 + b

          @pl.when(cix < count)
          def _():
            pltpu.make_async_copy(rows_v.at[b], acc_s.at[dst_v.at[cix]],
                                  sems[b]).wait()
            pltpu.async_copy(table.at[src_v.at[cix]], rows_v.at[b], semg[b])

      for b in range(2):
        pltpu.make_async_copy(rows_v.at[b], acc_s.at[dst_v.at[b]],
                              sems[b]).wait()
      if with_deg:
        def ddrain(i, carry):
          pltpu.make_async_copy(ones_v, deg_s.at[dst_v.at[i]], semd).wait()
          return carry
        lax.fori_loop(0, count, ddrain, 0)

    @pl.when(c == 0)
    def _():
      scatter_phase(s * nc0, nc0)

    @pl.when(c != 0)
    def _():
      scatter_phase(NS * nc0 + s * nc1, nc1)

    plsc.subcore_barrier()

    row0 = s * ROWS_PER_TILE
    pltpu.sync_copy(acc_s.at[pl.ds(row0, ROWS_PER_TILE)],
                    out_acc.at[c, pl.ds(row0, ROWS_PER_TILE)])
    if with_deg:
      pltpu.sync_copy(deg_s.at[pl.ds(row0, ROWS_PER_TILE)],
                      out_deg.at[c, pl.ds(row0, ROWS_PER_TILE)])

  return pl.kernel(body, out_type=out_type, mesh=mesh, scratch_types=scratch,
                   compiler_params=pltpu.CompilerParams(use_tc_tiling_on_sc=False))


def _gru_cell(xs2, h, wt, wh, b):
  g = (jnp.dot(xs2, wt, preferred_element_type=jnp.float32)
       + jnp.dot(h, wh, preferred_element_type=jnp.float32) + b)
  rz = jax.nn.sigmoid(g[:, :2 * HIDDEN])
  r = rz[:, :HIDDEN]
  zg = rz[:, HIDDEN:]
  n = jnp.tanh(g[:, 2 * HIDDEN:3 * HIDDEN] + r * g[:, 3 * HIDDEN:])
  return n + zg * (h - n)


def _gru_wcomb(w_ih, w_hh, b_ih, b_hh):
  wih, whh = w_ih.T, w_hh.T
  d = wih.shape[0]
  h2 = 2 * HIDDEN
  wih4 = jnp.concatenate(
      [wih[:, :h2], wih[:, h2:], jnp.zeros((d, HIDDEN), jnp.float32)], axis=1)
  eye = jnp.eye(T_SEQ, dtype=jnp.float32)
  wt = (eye[:, :, None, None] * wih4[None, None, :, :]).reshape(
      T_SEQ, T_SEQ * d, 4 * HIDDEN)
  wh = jnp.concatenate(
      [whh[:, :h2], jnp.zeros((HIDDEN, HIDDEN), jnp.float32), whh[:, h2:]],
      axis=1)
  b = jnp.concatenate([b_ih[:h2] + b_hh[:h2], b_ih[h2:], b_hh[h2:]])
  return wt, wh, b.reshape(1, -1)


def _pre_body(x_ref, w1_ref, table_ref, xr_ref):
  x = x_ref[...]
  y = jnp.dot(x, w1_ref[...], preferred_element_type=jnp.float32)
  table_ref[...] = y[:, :HIDDEN]
  xr_ref[...] = y[:, HIDDEN:]


def _gru_body(xs_ref, wtf_ref, whf_ref, bf_ref, wtb_ref, whb_ref, bb_ref,
              wfc_ref, bfc_ref, zmul_ref):
  xs2 = xs_ref[...]
  b = xs2.shape[0]
  hf = jnp.zeros((b, HIDDEN), jnp.float32)
  hb = jnp.zeros((b, HIDDEN), jnp.float32)
  whf, bf = whf_ref[...], bf_ref[...]
  whb, bb = whb_ref[...], bb_ref[...]
  for t in range(T_SEQ):
    hf = _gru_cell(xs2, hf, wtf_ref[t], whf, bf)
    hb = _gru_cell(xs2, hb, wtb_ref[T_SEQ - 1 - t], whb, bb)
  hcat = jnp.concatenate([hf, hb], axis=1)
  zmul_ref[...] = (jnp.dot(hcat, wfc_ref[...], preferred_element_type=jnp.float32)
                   + bfc_ref[...])


def _combine1_body(a0_ref, a1_ref, d0_ref, d1_ref, xr_ref, bl1_ref, w2_ref,
                   table2_ref, hr_ref):
  deg = d0_ref[...] + d1_ref[...]
  inv = 1.0 / jnp.maximum(deg, 1.0)
  h = jnp.maximum((a0_ref[...] + a1_ref[...]) * inv + bl1_ref[...] + xr_ref[...],
                  0.0)
  y = jnp.dot(h, w2_ref[...], preferred_element_type=jnp.float32)
  table2_ref[...] = y[:, :Z_DIM]
  hr_ref[...] = y[:, Z_DIM:]


def _fuse_body(a0_ref, a1_ref, d0_ref, d1_ref, hr_ref, bl2_ref, zmul_ref,
               wfus_ref, bfus_ref, cen_ref, z_ref, q_ref):
  deg = d0_ref[...] + d1_ref[...]
  inv = 1.0 / jnp.maximum(deg, 1.0)
  zg = (a0_ref[...] + a1_ref[...]) * inv + bl2_ref[...] + hr_ref[...]
  comb = jnp.concatenate([zg, zmul_ref[...]], axis=1)
  z = jnp.dot(comb, wfus_ref[...], preferred_element_type=jnp.float32) + bfus_ref[...]
  z_ref[...] = z
  cen = cen_ref[...]
  zc = lax.dot_general(z, cen, (((1,), (1,)), ((), ())),
                       preferred_element_type=jnp.float32)
  z2 = jnp.sum(z * z, axis=1, keepdims=True)
  c2 = jnp.sum(cen * cen, axis=1)[None, :]
  d2 = z2 + c2 - 2.0 * zc
  q = 1.0 / (1.0 + d2)
  q_ref[...] = q / jnp.sum(q, axis=1, keepdims=True)


def _adj_body(zi_ref, zj_ref, out_ref):
  out_ref[...] = jax.nn.sigmoid(
      lax.dot_general(zi_ref[...], zj_ref[...], (((1,), (1,)), ((), ())),
                      preferred_element_type=jnp.float32))


def kernel(x_static, edge_index, x_seq, Wl1, bl1, Wr1, Wl2, bl2, Wr2,
           W_ih_f, W_hh_f, b_ih_f, b_hh_f, W_ih_b, W_hh_b, b_ih_b, b_hh_b,
           W_fc, b_fc, W_fus, b_fus, centers):
  n = x_static.shape[0]
  e = edge_index.shape[1]
  ncl = centers.shape[0]
  grid = n // BLK

  w1cat = jnp.concatenate([Wl1, Wr1], axis=0).T
  w2cat = jnp.concatenate([Wl2, Wr2], axis=0).T
  src = edge_index[0].astype(jnp.int32)
  dst = edge_index[1].astype(jnp.int32)
  ct = -(-e // CHUNK)
  nc0 = max(2, 2 * round(ct * 0.62 / NS / 2))
  nc1 = max(2, 2 * (-(-max(ct - NS * nc0, 0) // (2 * NS))))
  ep = NS * (nc0 + nc1) * CHUNK
  src2 = jnp.concatenate([src, jnp.zeros((ep - e,), jnp.int32)]).reshape(
      -1, CHUNK)
  dst2 = jnp.concatenate([dst, jnp.full((ep - e,), n, jnp.int32)]).reshape(
      -1, CHUNK)

  full = lambda *shape: pl.BlockSpec(shape, lambda i: (0,) * len(shape))
  rowblk = lambda w: pl.BlockSpec((BLK, w), lambda i: (i, 0))
  partblk = lambda w: pl.BlockSpec((None, BLK, w), lambda i, _c=0: (_c, i, 0))

  table1, xr = pl.pallas_call(
      _pre_body,
      grid=(grid,),
      in_specs=[rowblk(128), full(128, 128)],
      out_specs=[rowblk(HIDDEN), rowblk(HIDDEN)],
      out_shape=[
          jax.ShapeDtypeStruct((n, HIDDEN), jnp.float32),
          jax.ShapeDtypeStruct((n, HIDDEN), jnp.float32),
      ],
  )(x_static, w1cat)

  agg1, degp = _make_segsum(HIDDEN, nc0, nc1, True)(table1, src2, dst2)
  degp = degp.reshape(NC, NP, 1)

  seq_d = x_seq.shape[2]
  wtf, whf, bgf = _gru_wcomb(W_ih_f, W_hh_f, b_ih_f, b_hh_f)
  wtb, whb, bgb = _gru_wcomb(W_ih_b, W_hh_b, b_ih_b, b_hh_b)
  td = T_SEQ * seq_d
  zmul = pl.pallas_call(
      _gru_body,
      grid=(grid,),
      in_specs=[
          rowblk(td),
          full(T_SEQ, td, 4 * HIDDEN), full(HIDDEN, 4 * HIDDEN),
          full(1, 4 * HIDDEN),
          full(T_SEQ, td, 4 * HIDDEN), full(HIDDEN, 4 * HIDDEN),
          full(1, 4 * HIDDEN),
          full(2 * HIDDEN, Z_DIM), full(1, Z_DIM),
      ],
      out_specs=rowblk(Z_DIM),
      out_shape=jax.ShapeDtypeStruct((n, Z_DIM), jnp.float32),
  )(x_seq.reshape(n, td), wtf, whf, bgf, wtb, whb, bgb,
    W_fc.T, b_fc.reshape(1, -1))

  part64 = [pl.BlockSpec((None, BLK, HIDDEN), lambda i: (0, i, 0)),
            pl.BlockSpec((None, BLK, HIDDEN), lambda i: (1, i, 0))]
  partd = [pl.BlockSpec((None, BLK, 1), lambda i: (0, i, 0)),
           pl.BlockSpec((None, BLK, 1), lambda i: (1, i, 0))]
  table2, hr = pl.pallas_call(
      _combine1_body,
      grid=(grid,),
      in_specs=part64 + partd + [rowblk(HIDDEN), full(1, HIDDEN),
                                 full(HIDDEN, HIDDEN)],
      out_specs=[rowblk(Z_DIM), rowblk(Z_DIM)],
      out_shape=[
          jax.ShapeDtypeStruct((n, Z_DIM), jnp.float32),
          jax.ShapeDtypeStruct((n, Z_DIM), jnp.float32),
      ],
  )(agg1, agg1, degp, degp, xr, bl1.reshape(1, -1), w2cat)

  (agg2,) = _make_segsum(Z_DIM, nc0, nc1, False)(table2, src2, dst2)

  part32 = [pl.BlockSpec((None, BLK, Z_DIM), lambda i: (0, i, 0)),
            pl.BlockSpec((None, BLK, Z_DIM), lambda i: (1, i, 0))]
  z, q = pl.pallas_call(
      _fuse_body,
      grid=(grid,),
      in_specs=part32 + partd + [rowblk(Z_DIM), full(1, Z_DIM), rowblk(Z_DIM),
                                 full(2 * Z_DIM, Z_DIM), full(1, Z_DIM),
                                 full(ncl, Z_DIM)],
      out_specs=[rowblk(Z_DIM), rowblk(ncl)],
      out_shape=[
          jax.ShapeDtypeStruct((n, Z_DIM), jnp.float32),
          jax.ShapeDtypeStruct((n, ncl), jnp.float32),
      ],
  )(agg2, agg2, degp, degp, hr, bl2.reshape(1, -1), zmul,
    W_fus.T, b_fus.reshape(1, -1), centers)

  adj = pl.pallas_call(
      _adj_body,
      grid=(n // ABLK,),
      in_specs=[pl.BlockSpec((ABLK, Z_DIM), lambda i: (i, 0)),
                pl.BlockSpec((n, Z_DIM), lambda i: (0, 0))],
      out_specs=pl.BlockSpec((ABLK, n), lambda i: (i, 0)),
      out_shape=jax.ShapeDtypeStruct((n, n), jnp.float32),
  )(z, z)

  return (z, adj, q)


# speedup vs baseline: 1.0151x; 1.0055x over previous
import functools

import jax
import jax.numpy as jnp
from jax import lax
from jax.experimental import pallas as pl
from jax.experimental.pallas import tpu as pltpu
from jax.experimental.pallas import tpu_sc as plsc

N_NODES = 10000
HIDDEN = 64
Z_DIM = 32
T_SEQ = 20

NC = 2
NS = 16
NW = NC * NS
CHUNK = 128
NP = 10240
ROWS_PER_TILE = NP // NS

BLK = 1000
ABLK = 400


def _make_segsum(width, nc0, nc1, with_deg):
  mesh = plsc.VectorSubcoreMesh(core_axis_name="c", subcore_axis_name="s")
  ncmax = max(nc0, nc1)

  out_type = [jax.ShapeDtypeStruct((NC, NP, width), jnp.float32)]
  scratch = [
      pltpu.VMEM((ncmax, CHUNK), jnp.int32),
      pltpu.VMEM((ncmax, CHUNK), jnp.int32),
      pltpu.VMEM((2, CHUNK, width), jnp.float32),
      pltpu.VMEM_SHARED((NP, width), jnp.float32),
      pltpu.SemaphoreType.DMA, pltpu.SemaphoreType.DMA,
      pltpu.SemaphoreType.DMA, pltpu.SemaphoreType.DMA,
  ]
  if with_deg:
    out_type.append(jax.ShapeDtypeStruct((NC, NP), jnp.float32))
    scratch += [
        pltpu.VMEM((CHUNK,), jnp.float32),
        pltpu.VMEM((ROWS_PER_TILE,), jnp.float32),
        pltpu.VMEM_SHARED((NP,), jnp.float32),
        pltpu.SemaphoreType.DMA,
    ]

  def body(table, src2, dst2, *rest):
    if with_deg:
      (out_acc, out_deg, src_v, dst_v, rows_v, acc_s, sg0, sg1, ss0, ss1,
       ones_v, zdeg_v, deg_s, semd) = rest
    else:
      out_acc, src_v, dst_v, rows_v, acc_s, sg0, sg1, ss0, ss1 = rest
    semg = (sg0, sg1)
    sems = (ss0, ss1)
    c = lax.axis_index("c")
    s = lax.axis_index("s")

    def zrow(i, carry):
      for j in range(width // 16):
        rows_v[0, i, pl.ds(j * 16, 16)] = jnp.zeros((16,), jnp.float32)
      return carry
    lax.fori_loop(0, CHUNK, zrow, 0)
    for i in range(ROWS_PER_TILE // CHUNK):
      pltpu.sync_copy(rows_v.at[0],
                      acc_s.at[pl.ds(s * ROWS_PER_TILE + i * CHUNK, CHUNK)])
    if with_deg:
      def zdeg(i, carry):
        zdeg_v[pl.ds(i * 16, 16)] = jnp.zeros((16,), jnp.float32)
        ones_v[pl.ds((i % (CHUNK // 16)) * 16, 16)] = jnp.ones((16,), jnp.float32)
        return carry
      lax.fori_loop(0, ROWS_PER_TILE // 16, zdeg, 0)
      pltpu.sync_copy(zdeg_v, deg_s.at[pl.ds(s * ROWS_PER_TILE, ROWS_PER_TILE)])
    plsc.subcore_barrier()

    def scatter_phase(base, count):
      pltpu.sync_copy(src2.at[pl.ds(base, count)], src_v.at[pl.ds(0, count)])
      pltpu.sync_copy(dst2.at[pl.ds(base, count)], dst_v.at[pl.ds(0, count)])

      for b in range(2):
        pltpu.async_copy(table.at[src_v.at[b]], rows_v.at[b], semg[b])

      @pl.loop(0, count, step=2)
      def chunk_body(i):
        for b in range(2):
          cix = i + b
          pltpu.make_async_copy(table.at[src_v.at[cix]], rows_v.at[b],
                                semg[b]).wait()
          pltpu.async_copy(rows_v.at[b], acc_s.at[dst_v.at[cix]], sems[b],
                           add=True)
          if with_deg:
            pltpu.async_copy(ones_v, deg_s.at[dst_v.at[cix]], semd, add=True)
        for b in range(2):
          cix = i + 2 + b

          @pl.when(cix < count)
          def _():
            pltpu.make_async_copy(rows_v.at[b], acc_s.at[dst_v.at[cix]],
                                  sems[b]).wait()
            pltpu.async_copy(table.at[src_v.at[cix]], rows_v.at[b], semg[b])

      for b in range(2):
        pltpu.make_async_copy(rows_v.at[b], acc_s.at[dst_v.at[b]],
                              sems[b]).wait()
      if with_deg:
        def ddrain(i, carry):
          pltpu.make_async_copy(ones_v, deg_s.at[dst_v.at[i]], semd).wait()
          return carry
        lax.fori_loop(0, count, ddrain, 0)

    @pl.when(c == 0)
    def _():
      scatter_phase(s * nc0, nc0)

    @pl.when(c != 0)
    def _():
      scatter_phase(NS * nc0 + s * nc1, nc1)

    plsc.subcore_barrier()

    row0 = s * ROWS_PER_TILE
    pltpu.sync_copy(acc_s.at[pl.ds(row0, ROWS_PER_TILE)],
                    out_acc.at[c, pl.ds(row0, ROWS_PER_TILE)])
    if with_deg:
      pltpu.sync_copy(deg_s.at[pl.ds(row0, ROWS_PER_TILE)],
                      out_deg.at[c, pl.ds(row0, ROWS_PER_TILE)])

  return pl.kernel(body, out_type=out_type, mesh=mesh, scratch_types=scratch,
                   compiler_params=pltpu.CompilerParams(use_tc_tiling_on_sc=False))


def _gru_cell(xs2, h, wt, wh, b):
  g = (jnp.dot(xs2, wt, preferred_element_type=jnp.float32)
       + jnp.dot(h, wh, preferred_element_type=jnp.float32) + b)
  rz = jax.nn.sigmoid(g[:, :2 * HIDDEN])
  r = rz[:, :HIDDEN]
  zg = rz[:, HIDDEN:]
  n = jnp.tanh(g[:, 2 * HIDDEN:3 * HIDDEN] + r * g[:, 3 * HIDDEN:])
  return n + zg * (h - n)


def _gru_wcomb(w_ih, w_hh, b_ih, b_hh):
  wih, whh = w_ih.T, w_hh.T
  d = wih.shape[0]
  h2 = 2 * HIDDEN
  wih4 = jnp.concatenate(
      [wih[:, :h2], wih[:, h2:], jnp.zeros((d, HIDDEN), jnp.float32)], axis=1)
  eye = jnp.eye(T_SEQ, dtype=jnp.float32)
  wt = (eye[:, :, None, None] * wih4[None, None, :, :]).reshape(
      T_SEQ, T_SEQ * d, 4 * HIDDEN)
  wh = jnp.concatenate(
      [whh[:, :h2], jnp.zeros((HIDDEN, HIDDEN), jnp.float32), whh[:, h2:]],
      axis=1)
  b = jnp.concatenate([b_ih[:h2] + b_hh[:h2], b_ih[h2:], b_hh[h2:]])
  return wt, wh, b.reshape(1, -1)


def _pre_body(x_ref, w1_ref, table_ref, xr_ref):
  x = x_ref[...]
  y = jnp.dot(x, w1_ref[...], preferred_element_type=jnp.float32)
  table_ref[...] = y[:, :HIDDEN]
  xr_ref[...] = y[:, HIDDEN:]


def _gru_body(xs_ref, wtf_ref, whf_ref, bf_ref, wtb_ref, whb_ref, bb_ref,
              wfc_ref, bfc_ref, zmul_ref):
  xs2 = xs_ref[...]
  b = xs2.shape[0]
  hf = jnp.zeros((b, HIDDEN), jnp.float32)
  hb = jnp.zeros((b, HIDDEN), jnp.float32)
  whf, bf = whf_ref[...], bf_ref[...]
  whb, bb = whb_ref[...], bb_ref[...]
  for t in range(T_SEQ):
    hf = _gru_cell(xs2, hf, wtf_ref[t], whf, bf)
    hb = _gru_cell(xs2, hb, wtb_ref[T_SEQ - 1 - t], whb, bb)
  hcat = jnp.concatenate([hf, hb], axis=1)
  zmul_ref[...] = (jnp.dot(hcat, wfc_ref[...], preferred_element_type=jnp.float32)
                   + bfc_ref[...])


def _combine1_body(a0_ref, a1_ref, d0_ref, d1_ref, xr_ref, bl1_ref, w2_ref,
                   table2_ref, hr_ref):
  deg = d0_ref[...] + d1_ref[...]
  inv = 1.0 / jnp.maximum(deg, 1.0)
  h = jnp.maximum((a0_ref[...] + a1_ref[...]) * inv + bl1_ref[...] + xr_ref[...],
                  0.0)
  y = jnp.dot(h, w2_ref[...], preferred_element_type=jnp.float32)
  table2_ref[...] = y[:, :Z_DIM]
  hr_ref[...] = y[:, Z_DIM:]


def _fuse_body(a0_ref, a1_ref, d0_ref, d1_ref, hr_ref, bl2_ref, zmul_ref,
               wfus_ref, bfus_ref, cen_ref, z_ref, q_ref):
  deg = d0_ref[...] + d1_ref[...]
  inv = 1.0 / jnp.maximum(deg, 1.0)
  zg = (a0_ref[...] + a1_ref[...]) * inv + bl2_ref[...] + hr_ref[...]
  comb = jnp.concatenate([zg, zmul_ref[...]], axis=1)
  z = jnp.dot(comb, wfus_ref[...], preferred_element_type=jnp.float32) + bfus_ref[...]
  z_ref[...] = z
  cen = cen_ref[...]
  zc = lax.dot_general(z, cen, (((1,), (1,)), ((), ())),
                       preferred_element_type=jnp.float32)
  z2 = jnp.sum(z * z, axis=1, keepdims=True)
  c2 = jnp.sum(cen * cen, axis=1)[None, :]
  d2 = z2 + c2 - 2.0 * zc
  q = 1.0 / (1.0 + d2)
  q_ref[...] = q / jnp.sum(q, axis=1, keepdims=True)


def _adj_body(zi_ref, zj_ref, out_ref):
  out_ref[...] = jax.nn.sigmoid(
      lax.dot_general(zi_ref[...], zj_ref[...], (((1,), (1,)), ((), ())),
                      preferred_element_type=jnp.float32))


def kernel(x_static, edge_index, x_seq, Wl1, bl1, Wr1, Wl2, bl2, Wr2,
           W_ih_f, W_hh_f, b_ih_f, b_hh_f, W_ih_b, W_hh_b, b_ih_b, b_hh_b,
           W_fc, b_fc, W_fus, b_fus, centers):
  n = x_static.shape[0]
  e = edge_index.shape[1]
  ncl = centers.shape[0]
  grid = n // BLK

  w1cat = jnp.concatenate([Wl1, Wr1], axis=0).T
  w2cat = jnp.concatenate([Wl2, Wr2], axis=0).T
  src = edge_index[0].astype(jnp.int32)
  dst = edge_index[1].astype(jnp.int32)
  ct = -(-e // CHUNK)
  nc0 = max(2, 2 * round(ct * 0.66 / NS / 2))
  nc1 = max(2, 2 * (-(-max(ct - NS * nc0, 0) // (2 * NS))))
  ep = NS * (nc0 + nc1) * CHUNK
  src2 = jnp.concatenate([src, jnp.zeros((ep - e,), jnp.int32)]).reshape(
      -1, CHUNK)
  dst2 = jnp.concatenate([dst, jnp.full((ep - e,), n, jnp.int32)]).reshape(
      -1, CHUNK)

  full = lambda *shape: pl.BlockSpec(shape, lambda i: (0,) * len(shape))
  rowblk = lambda w: pl.BlockSpec((BLK, w), lambda i: (i, 0))
  partblk = lambda w: pl.BlockSpec((None, BLK, w), lambda i, _c=0: (_c, i, 0))

  table1, xr = pl.pallas_call(
      _pre_body,
      grid=(grid,),
      in_specs=[rowblk(128), full(128, 128)],
      out_specs=[rowblk(HIDDEN), rowblk(HIDDEN)],
      out_shape=[
          jax.ShapeDtypeStruct((n, HIDDEN), jnp.float32),
          jax.ShapeDtypeStruct((n, HIDDEN), jnp.float32),
      ],
  )(x_static, w1cat)

  agg1, degp = _make_segsum(HIDDEN, nc0, nc1, True)(table1, src2, dst2)
  degp = degp.reshape(NC, NP, 1)

  seq_d = x_seq.shape[2]
  wtf, whf, bgf = _gru_wcomb(W_ih_f, W_hh_f, b_ih_f, b_hh_f)
  wtb, whb, bgb = _gru_wcomb(W_ih_b, W_hh_b, b_ih_b, b_hh_b)
  td = T_SEQ * seq_d
  zmul = pl.pallas_call(
      _gru_body,
      grid=(grid,),
      in_specs=[
          rowblk(td),
          full(T_SEQ, td, 4 * HIDDEN), full(HIDDEN, 4 * HIDDEN),
          full(1, 4 * HIDDEN),
          full(T_SEQ, td, 4 * HIDDEN), full(HIDDEN, 4 * HIDDEN),
          full(1, 4 * HIDDEN),
          full(2 * HIDDEN, Z_DIM), full(1, Z_DIM),
      ],
      out_specs=rowblk(Z_DIM),
      out_shape=jax.ShapeDtypeStruct((n, Z_DIM), jnp.float32),
  )(x_seq.reshape(n, td), wtf, whf, bgf, wtb, whb, bgb,
    W_fc.T, b_fc.reshape(1, -1))

  part64 = [pl.BlockSpec((None, BLK, HIDDEN), lambda i: (0, i, 0)),
            pl.BlockSpec((None, BLK, HIDDEN), lambda i: (1, i, 0))]
  partd = [pl.BlockSpec((None, BLK, 1), lambda i: (0, i, 0)),
           pl.BlockSpec((None, BLK, 1), lambda i: (1, i, 0))]
  table2, hr = pl.pallas_call(
      _combine1_body,
      grid=(grid,),
      in_specs=part64 + partd + [rowblk(HIDDEN), full(1, HIDDEN),
                                 full(HIDDEN, HIDDEN)],
      out_specs=[rowblk(Z_DIM), rowblk(Z_DIM)],
      out_shape=[
          jax.ShapeDtypeStruct((n, Z_DIM), jnp.float32),
          jax.ShapeDtypeStruct((n, Z_DIM), jnp.float32),
      ],
  )(agg1, agg1, degp, degp, xr, bl1.reshape(1, -1), w2cat)

  (agg2,) = _make_segsum(Z_DIM, nc0, nc1, False)(table2, src2, dst2)

  part32 = [pl.BlockSpec((None, BLK, Z_DIM), lambda i: (0, i, 0)),
            pl.BlockSpec((None, BLK, Z_DIM), lambda i: (1, i, 0))]
  z, q = pl.pallas_call(
      _fuse_body,
      grid=(grid,),
      in_specs=part32 + partd + [rowblk(Z_DIM), full(1, Z_DIM), rowblk(Z_DIM),
                                 full(2 * Z_DIM, Z_DIM), full(1, Z_DIM),
                                 full(ncl, Z_DIM)],
      out_specs=[rowblk(Z_DIM), rowblk(ncl)],
      out_shape=[
          jax.ShapeDtypeStruct((n, Z_DIM), jnp.float32),
          jax.ShapeDtypeStruct((n, ncl), jnp.float32),
      ],
  )(agg2, agg2, degp, degp, hr, bl2.reshape(1, -1), zmul,
    W_fus.T, b_fus.reshape(1, -1), centers)

  adj = pl.pallas_call(
      _adj_body,
      grid=(n // ABLK,),
      in_specs=[pl.BlockSpec((ABLK, Z_DIM), lambda i: (i, 0)),
                pl.BlockSpec((n, Z_DIM), lambda i: (0, 0))],
      out_specs=pl.BlockSpec((ABLK, n), lambda i: (i, 0)),
      out_shape=jax.ShapeDtypeStruct((n, n), jnp.float32),
  )(z, z)

  return (z, adj, q)


# speedup vs baseline: 1.0188x; 1.0036x over previous
import functools

import jax
import jax.numpy as jnp
from jax import lax
from jax.experimental import pallas as pl
from jax.experimental.pallas import tpu as pltpu
from jax.experimental.pallas import tpu_sc as plsc

N_NODES = 10000
HIDDEN = 64
Z_DIM = 32
T_SEQ = 20

NC = 2
NS = 16
NW = NC * NS
CHUNK = 128
NP = 10240
ROWS_PER_TILE = NP // NS

BLK = 1000
ABLK = 400


def _make_segsum(width, nc0, nc1, with_deg):
  mesh = plsc.VectorSubcoreMesh(core_axis_name="c", subcore_axis_name="s")
  ncmax = max(nc0, nc1)

  out_type = [jax.ShapeDtypeStruct((NC, NP, width), jnp.float32)]
  scratch = [
      pltpu.VMEM((ncmax, CHUNK), jnp.int32),
      pltpu.VMEM((ncmax, CHUNK), jnp.int32),
      pltpu.VMEM((2, CHUNK, width), jnp.float32),
      pltpu.VMEM_SHARED((NP, width), jnp.float32),
      pltpu.SemaphoreType.DMA, pltpu.SemaphoreType.DMA,
      pltpu.SemaphoreType.DMA, pltpu.SemaphoreType.DMA,
  ]
  if with_deg:
    out_type.append(jax.ShapeDtypeStruct((NC, NP), jnp.float32))
    scratch += [
        pltpu.VMEM((CHUNK,), jnp.float32),
        pltpu.VMEM((ROWS_PER_TILE,), jnp.float32),
        pltpu.VMEM_SHARED((NP,), jnp.float32),
        pltpu.SemaphoreType.DMA,
    ]

  def body(table, src2, dst2, *rest):
    if with_deg:
      (out_acc, out_deg, src_v, dst_v, rows_v, acc_s, sg0, sg1, ss0, ss1,
       ones_v, zdeg_v, deg_s, semd) = rest
    else:
      out_acc, src_v, dst_v, rows_v, acc_s, sg0, sg1, ss0, ss1 = rest
    semg = (sg0, sg1)
    sems = (ss0, ss1)
    c = lax.axis_index("c")
    s = lax.axis_index("s")

    def zrow(i, carry):
      for j in range(width // 16):
        rows_v[0, i, pl.ds(j * 16, 16)] = jnp.zeros((16,), jnp.float32)
      return carry
    lax.fori_loop(0, CHUNK, zrow, 0)
    for i in range(ROWS_PER_TILE // CHUNK):
      pltpu.sync_copy(rows_v.at[0],
                      acc_s.at[pl.ds(s * ROWS_PER_TILE + i * CHUNK, CHUNK)])
    if with_deg:
      def zdeg(i, carry):
        zdeg_v[pl.ds(i * 16, 16)] = jnp.zeros((16,), jnp.float32)
        ones_v[pl.ds((i % (CHUNK // 16)) * 16, 16)] = jnp.ones((16,), jnp.float32)
        return carry
      lax.fori_loop(0, ROWS_PER_TILE // 16, zdeg, 0)
      pltpu.sync_copy(zdeg_v, deg_s.at[pl.ds(s * ROWS_PER_TILE, ROWS_PER_TILE)])
    plsc.subcore_barrier()

    def scatter_phase(base, count):
      pltpu.sync_copy(src2.at[pl.ds(base, count)], src_v.at[pl.ds(0, count)])
      pltpu.sync_copy(dst2.at[pl.ds(base, count)], dst_v.at[pl.ds(0, count)])

      for b in range(2):
        pltpu.async_copy(table.at[src_v.at[b]], rows_v.at[b], semg[b])

      @pl.loop(0, count, step=2)
      def chunk_body(i):
        for b in range(2):
          cix = i + b
          pltpu.make_async_copy(table.at[src_v.at[cix]], rows_v.at[b],
                                semg[b]).wait()
          pltpu.async_copy(rows_v.at[b], acc_s.at[dst_v.at[cix]], sems[b],
                           add=True)
          if with_deg:
            pltpu.async_copy(ones_v, deg_s.at[dst_v.at[cix]], semd, add=True)
        for b in range(2):
          cix = i + 2 + b

          @pl.when(cix < count)
          def _():
            pltpu.make_async_copy(rows_v.at[b], acc_s.at[dst_v.at[cix]],
                                  sems[b]).wait()
            pltpu.async_copy(table.at[src_v.at[cix]], rows_v.at[b], semg[b])

      for b in range(2):
        pltpu.make_async_copy(rows_v.at[b], acc_s.at[dst_v.at[b]],
                              sems[b]).wait()
      if with_deg:
        def ddrain(i, carry):
          pltpu.make_async_copy(ones_v, deg_s.at[dst_v.at[i]], semd).wait()
          return carry
        lax.fori_loop(0, count, ddrain, 0)

    @pl.when(c == 0)
    def _():
      scatter_phase(s * nc0, nc0)

    @pl.when(c != 0)
    def _():
      scatter_phase(NS * nc0 + s * nc1, nc1)

    plsc.subcore_barrier()

    row0 = s * ROWS_PER_TILE
    pltpu.sync_copy(acc_s.at[pl.ds(row0, ROWS_PER_TILE)],
                    out_acc.at[c, pl.ds(row0, ROWS_PER_TILE)])
    if with_deg:
      pltpu.sync_copy(deg_s.at[pl.ds(row0, ROWS_PER_TILE)],
                      out_deg.at[c, pl.ds(row0, ROWS_PER_TILE)])

  return pl.kernel(body, out_type=out_type, mesh=mesh, scratch_types=scratch,
                   compiler_params=pltpu.CompilerParams(use_tc_tiling_on_sc=False))


def _gru_cell(xs2, h, wt, wh, b):
  g = (jnp.dot(xs2, wt, preferred_element_type=jnp.float32)
       + jnp.dot(h, wh, preferred_element_type=jnp.float32) + b)
  rz = jax.nn.sigmoid(g[:, :2 * HIDDEN])
  r = rz[:, :HIDDEN]
  zg = rz[:, HIDDEN:]
  n = jnp.tanh(g[:, 2 * HIDDEN:3 * HIDDEN] + r * g[:, 3 * HIDDEN:])
  return n + zg * (h - n)


def _gru_wcomb(w_ih, w_hh, b_ih, b_hh):
  wih, whh = w_ih.T, w_hh.T
  d = wih.shape[0]
  h2 = 2 * HIDDEN
  wih4 = jnp.concatenate(
      [wih[:, :h2], wih[:, h2:], jnp.zeros((d, HIDDEN), jnp.float32)], axis=1)
  eye = jnp.eye(T_SEQ, dtype=jnp.float32)
  wt = (eye[:, :, None, None] * wih4[None, None, :, :]).reshape(
      T_SEQ, T_SEQ * d, 4 * HIDDEN)
  wh = jnp.concatenate(
      [whh[:, :h2], jnp.zeros((HIDDEN, HIDDEN), jnp.float32), whh[:, h2:]],
      axis=1)
  b = jnp.concatenate([b_ih[:h2] + b_hh[:h2], b_ih[h2:], b_hh[h2:]])
  return wt, wh, b.reshape(1, -1)


def _pre_body(x_ref, w1_ref, table_ref, xr_ref):
  x = x_ref[...]
  y = jnp.dot(x, w1_ref[...], preferred_element_type=jnp.float32)
  table_ref[...] = y[:, :HIDDEN]
  xr_ref[...] = y[:, HIDDEN:]


def _gru_body(xs_ref, wtf_ref, whf_ref, bf_ref, wtb_ref, whb_ref, bb_ref,
              wfc_ref, bfc_ref, zmul_ref):
  xs2 = xs_ref[...]
  b = xs2.shape[0]
  hf = jnp.zeros((b, HIDDEN), jnp.float32)
  hb = jnp.zeros((b, HIDDEN), jnp.float32)
  whf, bf = whf_ref[...], bf_ref[...]
  whb, bb = whb_ref[...], bb_ref[...]
  for t in range(T_SEQ):
    hf = _gru_cell(xs2, hf, wtf_ref[t], whf, bf)
    hb = _gru_cell(xs2, hb, wtb_ref[T_SEQ - 1 - t], whb, bb)
  hcat = jnp.concatenate([hf, hb], axis=1)
  zmul_ref[...] = (jnp.dot(hcat, wfc_ref[...], preferred_element_type=jnp.float32)
                   + bfc_ref[...])


def _combine1_body(a0_ref, a1_ref, d0_ref, d1_ref, xr_ref, bl1_ref, w2_ref,
                   table2_ref, hr_ref):
  deg = d0_ref[...] + d1_ref[...]
  inv = 1.0 / jnp.maximum(deg, 1.0)
  h = jnp.maximum((a0_ref[...] + a1_ref[...]) * inv + bl1_ref[...] + xr_ref[...],
                  0.0)
  y = jnp.dot(h, w2_ref[...], preferred_element_type=jnp.float32)
  table2_ref[...] = y[:, :Z_DIM]
  hr_ref[...] = y[:, Z_DIM:]


def _fuse_body(a0_ref, a1_ref, d0_ref, d1_ref, hr_ref, bl2_ref, zmul_ref,
               wfus_ref, bfus_ref, cen_ref, z_ref, q_ref):
  deg = d0_ref[...] + d1_ref[...]
  inv = 1.0 / jnp.maximum(deg, 1.0)
  zg = (a0_ref[...] + a1_ref[...]) * inv + bl2_ref[...] + hr_ref[...]
  comb = jnp.concatenate([zg, zmul_ref[...]], axis=1)
  z = jnp.dot(comb, wfus_ref[...], preferred_element_type=jnp.float32) + bfus_ref[...]
  z_ref[...] = z
  cen = cen_ref[...]
  zc = lax.dot_general(z, cen, (((1,), (1,)), ((), ())),
                       preferred_element_type=jnp.float32)
  z2 = jnp.sum(z * z, axis=1, keepdims=True)
  c2 = jnp.sum(cen * cen, axis=1)[None, :]
  d2 = z2 + c2 - 2.0 * zc
  q = 1.0 / (1.0 + d2)
  q_ref[...] = q / jnp.sum(q, axis=1, keepdims=True)


def _adj_body(zi_ref, zj_ref, out_ref):
  out_ref[...] = jax.nn.sigmoid(
      lax.dot_general(zi_ref[...], zj_ref[...], (((1,), (1,)), ((), ())),
                      preferred_element_type=jnp.float32))


def kernel(x_static, edge_index, x_seq, Wl1, bl1, Wr1, Wl2, bl2, Wr2,
           W_ih_f, W_hh_f, b_ih_f, b_hh_f, W_ih_b, W_hh_b, b_ih_b, b_hh_b,
           W_fc, b_fc, W_fus, b_fus, centers):
  n = x_static.shape[0]
  e = edge_index.shape[1]
  ncl = centers.shape[0]
  grid = n // BLK

  w1cat = jnp.concatenate([Wl1, Wr1], axis=0).T
  w2cat = jnp.concatenate([Wl2, Wr2], axis=0).T
  src = edge_index[0].astype(jnp.int32)
  dst = edge_index[1].astype(jnp.int32)
  ct = -(-e // CHUNK)
  nc0 = max(2, 2 * round(ct * 0.72 / NS / 2))
  nc1 = max(2, 2 * (-(-max(ct - NS * nc0, 0) // (2 * NS))))
  ep = NS * (nc0 + nc1) * CHUNK
  src2 = jnp.concatenate([src, jnp.zeros((ep - e,), jnp.int32)]).reshape(
      -1, CHUNK)
  dst2 = jnp.concatenate([dst, jnp.full((ep - e,), n, jnp.int32)]).reshape(
      -1, CHUNK)

  full = lambda *shape: pl.BlockSpec(shape, lambda i: (0,) * len(shape))
  rowblk = lambda w: pl.BlockSpec((BLK, w), lambda i: (i, 0))
  partblk = lambda w: pl.BlockSpec((None, BLK, w), lambda i, _c=0: (_c, i, 0))

  table1, xr = pl.pallas_call(
      _pre_body,
      grid=(grid,),
      in_specs=[rowblk(128), full(128, 128)],
      out_specs=[rowblk(HIDDEN), rowblk(HIDDEN)],
      out_shape=[
          jax.ShapeDtypeStruct((n, HIDDEN), jnp.float32),
          jax.ShapeDtypeStruct((n, HIDDEN), jnp.float32),
      ],
  )(x_static, w1cat)

  agg1, degp = _make_segsum(HIDDEN, nc0, nc1, True)(table1, src2, dst2)
  degp = degp.reshape(NC, NP, 1)

  seq_d = x_seq.shape[2]
  wtf, whf, bgf = _gru_wcomb(W_ih_f, W_hh_f, b_ih_f, b_hh_f)
  wtb, whb, bgb = _gru_wcomb(W_ih_b, W_hh_b, b_ih_b, b_hh_b)
  td = T_SEQ * seq_d
  zmul = pl.pallas_call(
      _gru_body,
      grid=(grid,),
      in_specs=[
          rowblk(td),
          full(T_SEQ, td, 4 * HIDDEN), full(HIDDEN, 4 * HIDDEN),
          full(1, 4 * HIDDEN),
          full(T_SEQ, td, 4 * HIDDEN), full(HIDDEN, 4 * HIDDEN),
          full(1, 4 * HIDDEN),
          full(2 * HIDDEN, Z_DIM), full(1, Z_DIM),
      ],
      out_specs=rowblk(Z_DIM),
      out_shape=jax.ShapeDtypeStruct((n, Z_DIM), jnp.float32),
  )(x_seq.reshape(n, td), wtf, whf, bgf, wtb, whb, bgb,
    W_fc.T, b_fc.reshape(1, -1))

  part64 = [pl.BlockSpec((None, BLK, HIDDEN), lambda i: (0, i, 0)),
            pl.BlockSpec((None, BLK, HIDDEN), lambda i: (1, i, 0))]
  partd = [pl.BlockSpec((None, BLK, 1), lambda i: (0, i, 0)),
           pl.BlockSpec((None, BLK, 1), lambda i: (1, i, 0))]
  table2, hr = pl.pallas_call(
      _combine1_body,
      grid=(grid,),
      in_specs=part64 + partd + [rowblk(HIDDEN), full(1, HIDDEN),
                                 full(HIDDEN, HIDDEN)],
      out_specs=[rowblk(Z_DIM), rowblk(Z_DIM)],
      out_shape=[
          jax.ShapeDtypeStruct((n, Z_DIM), jnp.float32),
          jax.ShapeDtypeStruct((n, Z_DIM), jnp.float32),
      ],
  )(agg1, agg1, degp, degp, xr, bl1.reshape(1, -1), w2cat)

  (agg2,) = _make_segsum(Z_DIM, nc0, nc1, False)(table2, src2, dst2)

  part32 = [pl.BlockSpec((None, BLK, Z_DIM), lambda i: (0, i, 0)),
            pl.BlockSpec((None, BLK, Z_DIM), lambda i: (1, i, 0))]
  z, q = pl.pallas_call(
      _fuse_body,
      grid=(grid,),
      in_specs=part32 + partd + [rowblk(Z_DIM), full(1, Z_DIM), rowblk(Z_DIM),
                                 full(2 * Z_DIM, Z_DIM), full(1, Z_DIM),
                                 full(ncl, Z_DIM)],
      out_specs=[rowblk(Z_DIM), rowblk(ncl)],
      out_shape=[
          jax.ShapeDtypeStruct((n, Z_DIM), jnp.float32),
          jax.ShapeDtypeStruct((n, ncl), jnp.float32),
      ],
  )(agg2, agg2, degp, degp, hr, bl2.reshape(1, -1), zmul,
    W_fus.T, b_fus.reshape(1, -1), centers)

  adj = pl.pallas_call(
      _adj_body,
      grid=(n // ABLK,),
      in_specs=[pl.BlockSpec((ABLK, Z_DIM), lambda i: (i, 0)),
                pl.BlockSpec((n, Z_DIM), lambda i: (0, 0))],
      out_specs=pl.BlockSpec((ABLK, n), lambda i: (i, 0)),
      out_shape=jax.ShapeDtypeStruct((n, n), jnp.float32),
  )(z, z)

  return (z, adj, q)


# speedup vs baseline: 1.0195x; 1.0007x over previous
import functools

import jax
import jax.numpy as jnp
from jax import lax
from jax.experimental import pallas as pl
from jax.experimental.pallas import tpu as pltpu
from jax.experimental.pallas import tpu_sc as plsc

N_NODES = 10000
HIDDEN = 64
Z_DIM = 32
T_SEQ = 20

NC = 2
NS = 16
NW = NC * NS
CHUNK = 128
NP = 10240
ROWS_PER_TILE = NP // NS

BLK = 1000
ABLK = 400


def _make_segsum(width, nc0, nc1, with_deg):
  mesh = plsc.VectorSubcoreMesh(core_axis_name="c", subcore_axis_name="s")
  ncmax = max(nc0, nc1)

  out_type = [jax.ShapeDtypeStruct((NC, NP, width), jnp.float32)]
  scratch = [
      pltpu.VMEM((ncmax, CHUNK), jnp.int32),
      pltpu.VMEM((ncmax, CHUNK), jnp.int32),
      pltpu.VMEM((2, CHUNK, width), jnp.float32),
      pltpu.VMEM_SHARED((NP, width), jnp.float32),
      pltpu.SemaphoreType.DMA, pltpu.SemaphoreType.DMA,
      pltpu.SemaphoreType.DMA, pltpu.SemaphoreType.DMA,
  ]
  if with_deg:
    out_type.append(jax.ShapeDtypeStruct((NC, NP), jnp.float32))
    scratch += [
        pltpu.VMEM((CHUNK,), jnp.float32),
        pltpu.VMEM((ROWS_PER_TILE,), jnp.float32),
        pltpu.VMEM_SHARED((NP,), jnp.float32),
        pltpu.SemaphoreType.DMA,
    ]

  def body(table, src2, dst2, *rest):
    if with_deg:
      (out_acc, out_deg, src_v, dst_v, rows_v, acc_s, sg0, sg1, ss0, ss1,
       ones_v, zdeg_v, deg_s, semd) = rest
    else:
      out_acc, src_v, dst_v, rows_v, acc_s, sg0, sg1, ss0, ss1 = rest
    semg = (sg0, sg1)
    sems = (ss0, ss1)
    c = lax.axis_index("c")
    s = lax.axis_index("s")

    def zrow(i, carry):
      for j in range(width // 16):
        rows_v[0, i, pl.ds(j * 16, 16)] = jnp.zeros((16,), jnp.float32)
      return carry
    lax.fori_loop(0, CHUNK, zrow, 0)
    for i in range(ROWS_PER_TILE // CHUNK):
      pltpu.sync_copy(rows_v.at[0],
                      acc_s.at[pl.ds(s * ROWS_PER_TILE + i * CHUNK, CHUNK)])
    if with_deg:
      def zdeg(i, carry):
        zdeg_v[pl.ds(i * 16, 16)] = jnp.zeros((16,), jnp.float32)
        ones_v[pl.ds((i % (CHUNK // 16)) * 16, 16)] = jnp.ones((16,), jnp.float32)
        return carry
      lax.fori_loop(0, ROWS_PER_TILE // 16, zdeg, 0)
      pltpu.sync_copy(zdeg_v, deg_s.at[pl.ds(s * ROWS_PER_TILE, ROWS_PER_TILE)])
    plsc.subcore_barrier()

    def scatter_phase(base, count):
      pltpu.sync_copy(src2.at[pl.ds(base, count)], src_v.at[pl.ds(0, count)])
      pltpu.sync_copy(dst2.at[pl.ds(base, count)], dst_v.at[pl.ds(0, count)])

      for b in range(2):
        pltpu.async_copy(table.at[src_v.at[b]], rows_v.at[b], semg[b])

      @pl.loop(0, count, step=2)
      def chunk_body(i):
        for b in range(2):
          cix = i + b
          pltpu.make_async_copy(table.at[src_v.at[cix]], rows_v.at[b],
                                semg[b]).wait()
          pltpu.async_copy(rows_v.at[b], acc_s.at[dst_v.at[cix]], sems[b],
                           add=True)
          if with_deg:
            pltpu.async_copy(ones_v, deg_s.at[dst_v.at[cix]], semd, add=True)
        for b in range(2):
          cix = i + 2 + b

          @pl.when(cix < count)
          def _():
            pltpu.make_async_copy(rows_v.at[b], acc_s.at[dst_v.at[cix]],
                                  sems[b]).wait()
            pltpu.async_copy(table.at[src_v.at[cix]], rows_v.at[b], semg[b])

      for b in range(2):
        pltpu.make_async_copy(rows_v.at[b], acc_s.at[dst_v.at[b]],
                              sems[b]).wait()
      if with_deg:
        def ddrain(i, carry):
          pltpu.make_async_copy(ones_v, deg_s.at[dst_v.at[i]], semd).wait()
          return carry
        lax.fori_loop(0, count, ddrain, 0)

    @pl.when(c == 0)
    def _():
      scatter_phase(s * nc0, nc0)

    @pl.when(c != 0)
    def _():
      scatter_phase(NS * nc0 + s * nc1, nc1)

    plsc.subcore_barrier()

    row0 = s * ROWS_PER_TILE
    pltpu.sync_copy(acc_s.at[pl.ds(row0, ROWS_PER_TILE)],
                    out_acc.at[c, pl.ds(row0, ROWS_PER_TILE)])
    if with_deg:
      pltpu.sync_copy(deg_s.at[pl.ds(row0, ROWS_PER_TILE)],
                      out_deg.at[c, pl.ds(row0, ROWS_PER_TILE)])

  return pl.kernel(body, out_type=out_type, mesh=mesh, scratch_types=scratch,
                   compiler_params=pltpu.CompilerParams(use_tc_tiling_on_sc=False))


def _gru_cell(xs2, h, wt, wh, b):
  g = (jnp.dot(xs2, wt, preferred_element_type=jnp.float32)
       + jnp.dot(h, wh, preferred_element_type=jnp.float32) + b)
  rz = jax.nn.sigmoid(g[:, :2 * HIDDEN])
  r = rz[:, :HIDDEN]
  zg = rz[:, HIDDEN:]
  n = jnp.tanh(g[:, 2 * HIDDEN:3 * HIDDEN] + r * g[:, 3 * HIDDEN:])
  return n + zg * (h - n)


def _gru_wcomb(w_ih, w_hh, b_ih, b_hh):
  wih, whh = w_ih.T, w_hh.T
  d = wih.shape[0]
  h2 = 2 * HIDDEN
  wih4 = jnp.concatenate(
      [wih[:, :h2], wih[:, h2:], jnp.zeros((d, HIDDEN), jnp.float32)], axis=1)
  eye = jnp.eye(T_SEQ, dtype=jnp.float32)
  wt = (eye[:, :, None, None] * wih4[None, None, :, :]).reshape(
      T_SEQ, T_SEQ * d, 4 * HIDDEN)
  wh = jnp.concatenate(
      [whh[:, :h2], jnp.zeros((HIDDEN, HIDDEN), jnp.float32), whh[:, h2:]],
      axis=1)
  b = jnp.concatenate([b_ih[:h2] + b_hh[:h2], b_ih[h2:], b_hh[h2:]])
  return wt, wh, b.reshape(1, -1)


def _pre_body(x_ref, w1_ref, table_ref, xr_ref):
  x = x_ref[...]
  y = jnp.dot(x, w1_ref[...], preferred_element_type=jnp.float32)
  table_ref[...] = y[:, :HIDDEN]
  xr_ref[...] = y[:, HIDDEN:]


def _gru_body(xs_ref, wtf_ref, whf_ref, bf_ref, wtb_ref, whb_ref, bb_ref,
              wfc_ref, bfc_ref, zmul_ref):
  xs2 = xs_ref[...]
  b = xs2.shape[0]
  hf = jnp.zeros((b, HIDDEN), jnp.float32)
  hb = jnp.zeros((b, HIDDEN), jnp.float32)
  whf, bf = whf_ref[...], bf_ref[...]
  whb, bb = whb_ref[...], bb_ref[...]
  for t in range(T_SEQ):
    hf = _gru_cell(xs2, hf, wtf_ref[t], whf, bf)
    hb = _gru_cell(xs2, hb, wtb_ref[T_SEQ - 1 - t], whb, bb)
  hcat = jnp.concatenate([hf, hb], axis=1)
  zmul_ref[...] = (jnp.dot(hcat, wfc_ref[...], preferred_element_type=jnp.float32)
                   + bfc_ref[...])


def _combine1_body(a0_ref, a1_ref, d0_ref, d1_ref, xr_ref, bl1_ref, w2_ref,
                   table2_ref, hr_ref):
  deg = d0_ref[...] + d1_ref[...]
  inv = 1.0 / jnp.maximum(deg, 1.0)
  h = jnp.maximum((a0_ref[...] + a1_ref[...]) * inv + bl1_ref[...] + xr_ref[...],
                  0.0)
  y = jnp.dot(h, w2_ref[...], preferred_element_type=jnp.float32)
  table2_ref[...] = y[:, :Z_DIM]
  hr_ref[...] = y[:, Z_DIM:]


def _fuse_body(a0_ref, a1_ref, d0_ref, d1_ref, hr_ref, bl2_ref, zmul_ref,
               wfus_ref, bfus_ref, cen_ref, z_ref, q_ref):
  deg = d0_ref[...] + d1_ref[...]
  inv = 1.0 / jnp.maximum(deg, 1.0)
  zg = (a0_ref[...] + a1_ref[...]) * inv + bl2_ref[...] + hr_ref[...]
  comb = jnp.concatenate([zg, zmul_ref[...]], axis=1)
  z = jnp.dot(comb, wfus_ref[...], preferred_element_type=jnp.float32) + bfus_ref[...]
  z_ref[...] = z
  cen = cen_ref[...]
  zc = lax.dot_general(z, cen, (((1,), (1,)), ((), ())),
                       preferred_element_type=jnp.float32)
  z2 = jnp.sum(z * z, axis=1, keepdims=True)
  c2 = jnp.sum(cen * cen, axis=1)[None, :]
  d2 = z2 + c2 - 2.0 * zc
  q = 1.0 / (1.0 + d2)
  q_ref[...] = q / jnp.sum(q, axis=1, keepdims=True)


def _adj_body(zi_ref, zj_ref, out_ref):
  out_ref[...] = jax.nn.sigmoid(
      lax.dot_general(zi_ref[...], zj_ref[...], (((1,), (1,)), ((), ())),
                      preferred_element_type=jnp.float32))


def kernel(x_static, edge_index, x_seq, Wl1, bl1, Wr1, Wl2, bl2, Wr2,
           W_ih_f, W_hh_f, b_ih_f, b_hh_f, W_ih_b, W_hh_b, b_ih_b, b_hh_b,
           W_fc, b_fc, W_fus, b_fus, centers):
  n = x_static.shape[0]
  e = edge_index.shape[1]
  ncl = centers.shape[0]
  grid = n // BLK

  w1cat = jnp.concatenate([Wl1, Wr1], axis=0).T
  w2cat = jnp.concatenate([Wl2, Wr2], axis=0).T
  src = edge_index[0].astype(jnp.int32)
  dst = edge_index[1].astype(jnp.int32)
  ct = -(-e // CHUNK)
  nc0 = max(2, 2 * round(ct * 0.78 / NS / 2))
  nc1 = max(2, 2 * (-(-max(ct - NS * nc0, 0) // (2 * NS))))
  ep = NS * (nc0 + nc1) * CHUNK
  src2 = jnp.concatenate([src, jnp.zeros((ep - e,), jnp.int32)]).reshape(
      -1, CHUNK)
  dst2 = jnp.concatenate([dst, jnp.full((ep - e,), n, jnp.int32)]).reshape(
      -1, CHUNK)

  full = lambda *shape: pl.BlockSpec(shape, lambda i: (0,) * len(shape))
  rowblk = lambda w: pl.BlockSpec((BLK, w), lambda i: (i, 0))
  partblk = lambda w: pl.BlockSpec((None, BLK, w), lambda i, _c=0: (_c, i, 0))

  table1, xr = pl.pallas_call(
      _pre_body,
      grid=(grid,),
      in_specs=[rowblk(128), full(128, 128)],
      out_specs=[rowblk(HIDDEN), rowblk(HIDDEN)],
      out_shape=[
          jax.ShapeDtypeStruct((n, HIDDEN), jnp.float32),
          jax.ShapeDtypeStruct((n, HIDDEN), jnp.float32),
      ],
  )(x_static, w1cat)

  agg1, degp = _make_segsum(HIDDEN, nc0, nc1, True)(table1, src2, dst2)
  degp = degp.reshape(NC, NP, 1)

  seq_d = x_seq.shape[2]
  wtf, whf, bgf = _gru_wcomb(W_ih_f, W_hh_f, b_ih_f, b_hh_f)
  wtb, whb, bgb = _gru_wcomb(W_ih_b, W_hh_b, b_ih_b, b_hh_b)
  td = T_SEQ * seq_d
  zmul = pl.pallas_call(
      _gru_body,
      grid=(grid,),
      in_specs=[
          rowblk(td),
          full(T_SEQ, td, 4 * HIDDEN), full(HIDDEN, 4 * HIDDEN),
          full(1, 4 * HIDDEN),
          full(T_SEQ, td, 4 * HIDDEN), full(HIDDEN, 4 * HIDDEN),
          full(1, 4 * HIDDEN),
          full(2 * HIDDEN, Z_DIM), full(1, Z_DIM),
      ],
      out_specs=rowblk(Z_DIM),
      out_shape=jax.ShapeDtypeStruct((n, Z_DIM), jnp.float32),
  )(x_seq.reshape(n, td), wtf, whf, bgf, wtb, whb, bgb,
    W_fc.T, b_fc.reshape(1, -1))

  part64 = [pl.BlockSpec((None, BLK, HIDDEN), lambda i: (0, i, 0)),
            pl.BlockSpec((None, BLK, HIDDEN), lambda i: (1, i, 0))]
  partd = [pl.BlockSpec((None, BLK, 1), lambda i: (0, i, 0)),
           pl.BlockSpec((None, BLK, 1), lambda i: (1, i, 0))]
  table2, hr = pl.pallas_call(
      _combine1_body,
      grid=(grid,),
      in_specs=part64 + partd + [rowblk(HIDDEN), full(1, HIDDEN),
                                 full(HIDDEN, HIDDEN)],
      out_specs=[rowblk(Z_DIM), rowblk(Z_DIM)],
      out_shape=[
          jax.ShapeDtypeStruct((n, Z_DIM), jnp.float32),
          jax.ShapeDtypeStruct((n, Z_DIM), jnp.float32),
      ],
  )(agg1, agg1, degp, degp, xr, bl1.reshape(1, -1), w2cat)

  (agg2,) = _make_segsum(Z_DIM, nc0, nc1, False)(table2, src2, dst2)

  part32 = [pl.BlockSpec((None, BLK, Z_DIM), lambda i: (0, i, 0)),
            pl.BlockSpec((None, BLK, Z_DIM), lambda i: (1, i, 0))]
  z, q = pl.pallas_call(
      _fuse_body,
      grid=(grid,),
      in_specs=part32 + partd + [rowblk(Z_DIM), full(1, Z_DIM), rowblk(Z_DIM),
                                 full(2 * Z_DIM, Z_DIM), full(1, Z_DIM),
                                 full(ncl, Z_DIM)],
      out_specs=[rowblk(Z_DIM), rowblk(ncl)],
      out_shape=[
          jax.ShapeDtypeStruct((n, Z_DIM), jnp.float32),
          jax.ShapeDtypeStruct((n, ncl), jnp.float32),
      ],
  )(agg2, agg2, degp, degp, hr, bl2.reshape(1, -1), zmul,
    W_fus.T, b_fus.reshape(1, -1), centers)

  adj = pl.pallas_call(
      _adj_body,
      grid=(n // ABLK,),
      in_specs=[pl.BlockSpec((ABLK, Z_DIM), lambda i: (i, 0)),
                pl.BlockSpec((n, Z_DIM), lambda i: (0, 0))],
      out_specs=pl.BlockSpec((ABLK, n), lambda i: (i, 0)),
      out_shape=jax.ShapeDtypeStruct((n, n), jnp.float32),
  )(z, z)

  return (z, adj, q)
